# baseline (device time: 54328 ns/iter reference)
import jax
import jax.numpy as jnp
from jax import lax
from jax.experimental import pallas as pl
from jax.experimental.pallas import tpu as pltpu

N_DEV = 4
FP8 = jnp.float8_e5m2


def kernel(x, w_mat, scale_x, scale_w):
    if x.dtype != FP8:
        x = x.astype(FP8)
    if w_mat.dtype != FP8:
        w_mat = w_mat.astype(FP8)

    m_glob, k_shard = x.shape
    m_blk = m_glob // N_DEV
    n = w_mat.shape[1]

    def body(x_ref, w_ref, sx_ref, sw_ref, out_ref,
             comm_ref, send_sems, recv_sems):
        my = lax.axis_index("i")

        barrier = pltpu.get_barrier_semaphore()
        for d in range(1, N_DEV):
            pl.semaphore_signal(
                barrier, inc=1,
                device_id=((my + d) % N_DEV,),
                device_id_type=pl.DeviceIdType.MESH,
            )
        pl.semaphore_wait(barrier, N_DEV - 1)

        sends = []
        for d in range(1, N_DEV):
            tgt = (my + d) % N_DEV
            rdma = pltpu.make_async_remote_copy(
                src_ref=x_ref.at[pl.ds(tgt * m_blk, m_blk), :],
                dst_ref=comm_ref.at[my],
                send_sem=send_sems.at[d - 1],
                recv_sem=recv_sems.at[my],
                device_id=(tgt,),
                device_id_type=pl.DeviceIdType.MESH,
            )
            rdma.start()
            sends.append(rdma)

        scale = sx_ref[0] * sw_ref[0]
        acc = lax.dot(
            x_ref[pl.ds(my * m_blk, m_blk), :],
            w_ref[pl.ds(my * k_shard, k_shard), :],
            preferred_element_type=jnp.float32,
        )

        for d in range(1, N_DEV):
            src = (my + N_DEV - d) % N_DEV
            recv = pltpu.make_async_remote_copy(
                src_ref=x_ref.at[pl.ds(0, m_blk), :],
                dst_ref=comm_ref.at[src],
                send_sem=send_sems.at[d - 1],
                recv_sem=recv_sems.at[src],
                device_id=(src,),
                device_id_type=pl.DeviceIdType.MESH,
            )
            recv.wait_recv()
            acc += lax.dot(
                comm_ref[src],
                w_ref[pl.ds(src * k_shard, k_shard), :],
                preferred_element_type=jnp.float32,
            )

        out_ref[:, :] = jnp.maximum(acc * scale, 0.0)

        for s in sends:
            s.wait_send()

    return pl.pallas_call(
        body,
        out_shape=jax.ShapeDtypeStruct((m_blk, n), jnp.float32),
        in_specs=[
            pl.BlockSpec(memory_space=pltpu.VMEM),
            pl.BlockSpec(memory_space=pltpu.VMEM),
            pl.BlockSpec(memory_space=pltpu.SMEM),
            pl.BlockSpec(memory_space=pltpu.SMEM),
        ],
        out_specs=pl.BlockSpec(memory_space=pltpu.VMEM),
        scratch_shapes=[
            pltpu.VMEM((N_DEV, m_blk, k_shard), FP8),
            pltpu.SemaphoreType.DMA((N_DEV - 1,)),
            pltpu.SemaphoreType.DMA((N_DEV,)),
        ],
        compiler_params=pltpu.CompilerParams(collective_id=0),
    )(x, w_mat, scale_x, scale_w)


# device time: 44413 ns/iter; 1.2232x vs baseline; 1.2232x over previous
import jax
import jax.numpy as jnp
from jax import lax
from jax.experimental import pallas as pl
from jax.experimental.pallas import tpu as pltpu

N_DEV = 4
FP8 = jnp.float8_e5m2


def kernel(x, w_mat, scale_x, scale_w):
    if x.dtype != jnp.float32:
        x = x.astype(jnp.float32)
    if w_mat.dtype != jnp.float32:
        w_mat = w_mat.astype(jnp.float32)

    m_glob, k_shard = x.shape
    m_blk = m_glob // N_DEV
    n = w_mat.shape[1]

    def body(x_hbm, w_hbm, sx_ref, sw_ref, out_ref,
             xbuf, xq_ref, wbuf, wq_ref, comm_ref,
             xsems, wsems, send_sems, recv_sems):
        my = lax.axis_index("i")

        barrier = pltpu.get_barrier_semaphore()
        for d in range(1, N_DEV):
            pl.semaphore_signal(
                barrier, inc=1,
                device_id=((my + d) % N_DEV,),
                device_id_type=pl.DeviceIdType.MESH,
            )

        xdmas = []
        for r in range(N_DEV):
            j = (my + r + 1) % N_DEV
            dma = pltpu.make_async_copy(
                x_hbm.at[pl.ds(j * m_blk, m_blk), :], xbuf.at[r], xsems.at[r]
            )
            dma.start()
            xdmas.append(dma)

        b_order = [my] + [(my + N_DEV - d) % N_DEV for d in range(1, N_DEV)]

        def start_w(stage, buf):
            dma = pltpu.make_async_copy(
                w_hbm.at[pl.ds(b_order[stage] * k_shard, k_shard), :],
                wbuf.at[buf], wsems.at[buf],
            )
            dma.start()
            return dma

        wdma = [start_w(0, 0), start_w(1, 1)]

        pl.semaphore_wait(barrier, N_DEV - 1)
        sends = []
        for r in range(N_DEV - 1):
            tgt = (my + r + 1) % N_DEV
            xdmas[r].wait()
            xq_ref[r] = xbuf[r].astype(FP8)
            rdma = pltpu.make_async_remote_copy(
                src_ref=xq_ref.at[r],
                dst_ref=comm_ref.at[my],
                send_sem=send_sems.at[r],
                recv_sem=recv_sems.at[my],
                device_id=(tgt,),
                device_id_type=pl.DeviceIdType.MESH,
            )
            rdma.start()
            sends.append(rdma)
        xdmas[3].wait()
        xq_ref[3] = xbuf[3].astype(FP8)

        pending = {0: wdma[0], 1: wdma[1]}
        pending[0].wait()
        wq_ref[0] = wbuf[0].astype(FP8)
        pending[0] = start_w(2, 0)
        out_ref[...] = lax.dot(
            xq_ref[3], wq_ref[0], preferred_element_type=jnp.float32
        )

        for d in range(1, N_DEV):
            src = (my + N_DEV - d) % N_DEV
            buf = d % 2
            pending[buf].wait()
            wq_ref[buf] = wbuf[buf].astype(FP8)
            if d + 2 < N_DEV:
                pending[buf] = start_w(d + 2, buf)
            recv = pltpu.make_async_remote_copy(
                src_ref=xq_ref.at[0],
                dst_ref=comm_ref.at[src],
                send_sem=send_sems.at[0],
                recv_sem=recv_sems.at[src],
                device_id=(src,),
                device_id_type=pl.DeviceIdType.MESH,
            )
            recv.wait_recv()
            out_ref[...] += lax.dot(
                comm_ref[src], wq_ref[buf], preferred_element_type=jnp.float32
            )

        scale = sx_ref[0] * sw_ref[0]
        out_ref[...] = jnp.maximum(out_ref[...] * scale, 0.0)

        for s in sends:
            s.wait_send()

    return pl.pallas_call(
        body,
        out_shape=jax.ShapeDtypeStruct((m_blk, n), jnp.float32),
        in_specs=[
            pl.BlockSpec(memory_space=pl.ANY),
            pl.BlockSpec(memory_space=pl.ANY),
            pl.BlockSpec(memory_space=pltpu.SMEM),
            pl.BlockSpec(memory_space=pltpu.SMEM),
        ],
        out_specs=pl.BlockSpec(memory_space=pltpu.VMEM),
        scratch_shapes=[
            pltpu.VMEM((N_DEV, m_blk, k_shard), jnp.float32),
            pltpu.VMEM((N_DEV, m_blk, k_shard), FP8),
            pltpu.VMEM((2, k_shard, n), jnp.float32),
            pltpu.VMEM((2, k_shard, n), FP8),
            pltpu.VMEM((N_DEV, m_blk, k_shard), FP8),
            pltpu.SemaphoreType.DMA((N_DEV,)),
            pltpu.SemaphoreType.DMA((2,)),
            pltpu.SemaphoreType.DMA((N_DEV - 1,)),
            pltpu.SemaphoreType.DMA((N_DEV,)),
        ],
        compiler_params=pltpu.CompilerParams(
            collective_id=0,
            vmem_limit_bytes=60 * 1024 * 1024,
        ),
    )(x, w_mat, scale_x, scale_w)
